# baseline (speedup 1.0000x reference)
import jax
import jax.numpy as jnp
from jax import lax
from jax.experimental import pallas as pl
from jax.experimental.pallas import tpu as pltpu
from jax.experimental.pallas import tpu_sc as plsc

_N = 10000
_E = 320000
_D = 128
_NC = 2
_NS = 16
_CHUNK = 128
_NT0 = 96
_NT1 = 64
_ROWS = _NS * (_NT0 + _NT1)
_EP = _ROWS * _CHUNK
_NR = 10112
_RPT = _NR // _NS
_NB = 2
_IB = 16


def _sc_agg_body(with_deg, src_hbm, dst_hbm, p_hbm, out_hbm, deg_hbm,
                 sblk, dblk, bufs, sems, acc, ones, dacc, dbuf):
    cid = lax.axis_index("c")
    sid = lax.axis_index("s")
    is0 = cid == 0
    ntc = jnp.where(is0, _NT0, _NT1)
    nblk = ntc // _IB
    row_base = jnp.where(is0, sid * _NT0, _NS * _NT0 + sid * _NT1)
    base = sid * _RPT
    rows = bufs[0]

    def zrow(i, c):
        for k in range(_D // 16):
            rows[i, pl.ds(k * 16, 16)] = jnp.zeros((16,), jnp.float32)
        return c
    lax.fori_loop(0, _CHUNK, zrow, 0)
    nfull = _RPT // _CHUNK
    for t in range(nfull):
        pltpu.sync_copy(rows, acc.at[pl.ds(base + t * _CHUNK, _CHUNK)])
    rem = _RPT - nfull * _CHUNK
    if rem:
        pltpu.sync_copy(rows.at[pl.ds(0, rem)],
                        acc.at[pl.ds(base + nfull * _CHUNK, rem)])
    if with_deg:
        for k in range(_CHUNK // 16):
            ones[pl.ds(k * 16, 16)] = jnp.ones((16,), jnp.float32)
        for k in range(_RPT // 16):
            dbuf[pl.ds(k * 16, 16)] = jnp.zeros((16,), jnp.float32)
        dbuf[pl.ds(_RPT - 16, 16)] = jnp.zeros((16,), jnp.float32)
        pltpu.sync_copy(dbuf, dacc.at[pl.ds(base, _RPT)])
    plsc.subcore_barrier()

    pltpu.sync_copy(src_hbm.at[pl.ds(row_base, _IB)], sblk.at[0])
    for b in range(_NB):
        pltpu.async_copy(p_hbm.at[sblk.at[0, b]], bufs[b], sems[b])

    def block(blk, c):
        @pl.when(blk + 1 < nblk)
        def _():
            pltpu.sync_copy(
                src_hbm.at[pl.ds(row_base + (blk + 1) * _IB, _IB)],
                sblk.at[(blk + 1) % 2])
        pltpu.sync_copy(dst_hbm.at[pl.ds(row_base + blk * _IB, _IB)], dblk)

        def group(g, c2):
            for b in range(_NB):
                j = blk * _IB + g * _NB + b
                pltpu.make_async_copy(p_hbm.at[pl.ds(0, _CHUNK)],
                                      bufs[b], sems[b]).wait()
                pltpu.sync_copy(bufs[b], acc.at[dblk.at[g * _NB + b]],
                                add=True)
                if with_deg:
                    pltpu.sync_copy(ones, dacc.at[dblk.at[g * _NB + b]],
                                    add=True)
                jn = j + _NB

                @pl.when(jn < ntc)
                def _():
                    pltpu.async_copy(
                        p_hbm.at[sblk.at[(jn // _IB) % 2, jn % _IB]],
                        bufs[b], sems[b])
            return c2
        lax.fori_loop(0, _IB // _NB, group, 0)
        return c
    lax.fori_loop(0, nblk, block, 0)
    plsc.subcore_barrier()

    nout = -(-_RPT // _CHUNK)
    for t in range(nout):
        rn = min(_CHUNK, _RPT - t * _CHUNK)
        b = t % _NB
        pltpu.sync_copy(acc.at[pl.ds(base + t * _CHUNK, rn)],
                        bufs[b].at[pl.ds(0, rn)])
        pltpu.sync_copy(bufs[b].at[pl.ds(0, rn)],
                        out_hbm.at[cid, pl.ds(base + t * _CHUNK, rn)])
    if with_deg:
        pltpu.sync_copy(dacc.at[pl.ds(base, _RPT)], dbuf)
        pltpu.sync_copy(dbuf, deg_hbm.at[pl.ds(cid * _NR + base, _RPT)])


def _make_sc_agg(with_deg):
    mesh = plsc.VectorSubcoreMesh(core_axis_name="c", subcore_axis_name="s")
    outs = [jax.ShapeDtypeStruct((_NC, _NR, _D), jnp.float32)]
    scratch = [
        pltpu.VMEM((2, _IB, _CHUNK), jnp.int32),
        pltpu.VMEM((_IB, _CHUNK), jnp.int32),
        [pltpu.VMEM((_CHUNK, _D), jnp.float32) for _ in range(_NB)],
        [pltpu.SemaphoreType.DMA for _ in range(_NB)],
        pltpu.VMEM_SHARED((_NR, _D), jnp.float32),
    ]
    if with_deg:
        outs.append(jax.ShapeDtypeStruct((_NC * _NR,), jnp.float32))
        scratch += [
            pltpu.VMEM((_CHUNK,), jnp.float32),
            pltpu.VMEM_SHARED((_NR,), jnp.float32),
            pltpu.VMEM((_RPT,), jnp.float32),
        ]

        def body(src_hbm, dst_hbm, p_hbm, out_hbm, deg_hbm,
                 sblk, dblk, bufs, sems, acc, ones, dacc, dbuf):
            _sc_agg_body(True, src_hbm, dst_hbm, p_hbm, out_hbm, deg_hbm,
                         sblk, dblk, bufs, sems, acc, ones, dacc, dbuf)
    else:
        def body(src_hbm, dst_hbm, p_hbm, out_hbm,
                 sblk, dblk, bufs, sems, acc):
            _sc_agg_body(False, src_hbm, dst_hbm, p_hbm, out_hbm, None,
                         sblk, dblk, bufs, sems, acc, None, None, None)

    return pl.kernel(body, out_type=tuple(outs), mesh=mesh,
                     scratch_types=scratch,
                     name="sc_agg_deg" if with_deg else "sc_agg")


_sc_agg_deg = _make_sc_agg(True)
_sc_agg = _make_sc_agg(False)


def _dense_in_body(x_ref, we, be, wl, bl, wr, p_ref, r_ref):
    h = jnp.dot(x_ref[...], we[...], preferred_element_type=jnp.float32)
    h = h + be[...]
    p_ref[...] = jnp.dot(h, wl[...], preferred_element_type=jnp.float32)
    r_ref[...] = jnp.dot(h, wr[...], preferred_element_type=jnp.float32) + bl[...]


def _bn_relu(a0, a1, d0, d1, r, g, b):
    deg = jnp.maximum(d0[...] + d1[...], 1.0)
    u = (a0[...] + a1[...]) / deg + r[...]
    mean = jnp.mean(u, axis=0, keepdims=True)
    var = jnp.mean((u - mean) ** 2, axis=0, keepdims=True)
    hn = (u - mean) * lax.rsqrt(var + 1e-5) * g[...] + b[...]
    return jnp.maximum(hn, 0.0)


def _mid_body(a0, a1, d0, d1, r, g, b, wl, bl, wr, p_ref, r_ref):
    h = _bn_relu(a0, a1, d0, d1, r, g, b)
    p_ref[...] = jnp.dot(h, wl[...], preferred_element_type=jnp.float32)
    r_ref[...] = jnp.dot(h, wr[...], preferred_element_type=jnp.float32) + bl[...]


def _out_body(a0, a1, d0, d1, r, g, b, wc, bc, o_ref):
    h = _bn_relu(a0, a1, d0, d1, r, g, b)
    o_ref[...] = jnp.dot(h, wc[...], preferred_element_type=jnp.float32) + bc[...]


_f = jnp.float32
_dense_in = pl.pallas_call(
    _dense_in_body,
    out_shape=(jax.ShapeDtypeStruct((_N, _D), _f),
               jax.ShapeDtypeStruct((_N, _D), _f)))
_mid = pl.pallas_call(
    _mid_body,
    out_shape=(jax.ShapeDtypeStruct((_N, _D), _f),
               jax.ShapeDtypeStruct((_N, _D), _f)))
_outk = pl.pallas_call(
    _out_body,
    out_shape=jax.ShapeDtypeStruct((_N, 40), _f))


def kernel(x, edge_index, W_emb, b_emb, Wl1, bl1, Wr1, gamma1, beta1,
           Wl2, bl2, Wr2, gamma2, beta2, Wl3, bl3, Wr3, gamma3, beta3,
           W_cls, b_cls):
    src = edge_index[0].astype(jnp.int32)
    dst = edge_index[1].astype(jnp.int32)
    pad = _EP - _E
    src_p = jnp.concatenate([src, jnp.zeros((pad,), jnp.int32)])
    dst_p = jnp.concatenate(
        [dst, _N + (jnp.arange(pad, dtype=jnp.int32) % (_NR - _N))])
    src_p = src_p.reshape(_ROWS, _CHUNK)
    dst_p = dst_p.reshape(_ROWS, _CHUNK)

    p1, r1 = _dense_in(x, W_emb, b_emb, Wl1, bl1, Wr1)
    parts1, degp = _sc_agg_deg(src_p, dst_p, p1)
    d0 = degp[:_N].reshape(_N, 1)
    d1 = degp[_NR:_NR + _N].reshape(_N, 1)

    p2, r2 = _mid(parts1[0, :_N], parts1[1, :_N], d0, d1, r1,
                  gamma1, beta1, Wl2, bl2, Wr2)
    parts2, = _sc_agg(src_p, dst_p, p2)
    p3, r3 = _mid(parts2[0, :_N], parts2[1, :_N], d0, d1, r2,
                  gamma2, beta2, Wl3, bl3, Wr3)
    parts3, = _sc_agg(src_p, dst_p, p3)
    logits = _outk(parts3[0, :_N], parts3[1, :_N], d0, d1, r3,
                   gamma3, beta3, W_cls, b_cls)
    return logits


# speedup vs baseline: 1.0206x; 1.0206x over previous
import jax
import jax.numpy as jnp
from jax import lax
from jax.experimental import pallas as pl
from jax.experimental.pallas import tpu as pltpu
from jax.experimental.pallas import tpu_sc as plsc

_N = 10000
_E = 320000
_D = 128
_NC = 2
_NS = 16
_CHUNK = 128
_NT0 = 112
_NT1 = 48
_ROWS = _NS * (_NT0 + _NT1)
_EP = _ROWS * _CHUNK
_NR = 10112
_RPT = _NR // _NS
_NB = 2
_IB = 16


def _sc_agg_body(with_deg, src_hbm, dst_hbm, p_hbm, out_hbm, deg_hbm,
                 sblk, dblk, bufs, sems, acc, ones, dacc, dbuf):
    cid = lax.axis_index("c")
    sid = lax.axis_index("s")
    is0 = cid == 0
    ntc = jnp.where(is0, _NT0, _NT1)
    nblk = ntc // _IB
    row_base = jnp.where(is0, sid * _NT0, _NS * _NT0 + sid * _NT1)
    base = sid * _RPT
    rows = bufs[0]

    def zrow(i, c):
        for k in range(_D // 16):
            rows[i, pl.ds(k * 16, 16)] = jnp.zeros((16,), jnp.float32)
        return c
    lax.fori_loop(0, _CHUNK, zrow, 0)
    nfull = _RPT // _CHUNK
    for t in range(nfull):
        pltpu.sync_copy(rows, acc.at[pl.ds(base + t * _CHUNK, _CHUNK)])
    rem = _RPT - nfull * _CHUNK
    if rem:
        pltpu.sync_copy(rows.at[pl.ds(0, rem)],
                        acc.at[pl.ds(base + nfull * _CHUNK, rem)])
    if with_deg:
        for k in range(_CHUNK // 16):
            ones[pl.ds(k * 16, 16)] = jnp.ones((16,), jnp.float32)
        for k in range(_RPT // 16):
            dbuf[pl.ds(k * 16, 16)] = jnp.zeros((16,), jnp.float32)
        dbuf[pl.ds(_RPT - 16, 16)] = jnp.zeros((16,), jnp.float32)
        pltpu.sync_copy(dbuf, dacc.at[pl.ds(base, _RPT)])
    plsc.subcore_barrier()

    pltpu.sync_copy(src_hbm.at[pl.ds(row_base, _IB)], sblk.at[0])
    for b in range(_NB):
        pltpu.async_copy(p_hbm.at[sblk.at[0, b]], bufs[b], sems[b])

    def block(blk, c):
        @pl.when(blk + 1 < nblk)
        def _():
            pltpu.sync_copy(
                src_hbm.at[pl.ds(row_base + (blk + 1) * _IB, _IB)],
                sblk.at[(blk + 1) % 2])
        pltpu.sync_copy(dst_hbm.at[pl.ds(row_base + blk * _IB, _IB)], dblk)

        def group(g, c2):
            for b in range(_NB):
                j = blk * _IB + g * _NB + b
                pltpu.make_async_copy(p_hbm.at[pl.ds(0, _CHUNK)],
                                      bufs[b], sems[b]).wait()
                pltpu.sync_copy(bufs[b], acc.at[dblk.at[g * _NB + b]],
                                add=True)
                if with_deg:
                    pltpu.sync_copy(ones, dacc.at[dblk.at[g * _NB + b]],
                                    add=True)
                jn = j + _NB

                @pl.when(jn < ntc)
                def _():
                    pltpu.async_copy(
                        p_hbm.at[sblk.at[(jn // _IB) % 2, jn % _IB]],
                        bufs[b], sems[b])
            return c2
        lax.fori_loop(0, _IB // _NB, group, 0)
        return c
    lax.fori_loop(0, nblk, block, 0)
    plsc.subcore_barrier()

    nout = -(-_RPT // _CHUNK)
    for t in range(nout):
        rn = min(_CHUNK, _RPT - t * _CHUNK)
        b = t % _NB
        pltpu.sync_copy(acc.at[pl.ds(base + t * _CHUNK, rn)],
                        bufs[b].at[pl.ds(0, rn)])
        pltpu.sync_copy(bufs[b].at[pl.ds(0, rn)],
                        out_hbm.at[cid, pl.ds(base + t * _CHUNK, rn)])
    if with_deg:
        pltpu.sync_copy(dacc.at[pl.ds(base, _RPT)], dbuf)
        pltpu.sync_copy(dbuf, deg_hbm.at[pl.ds(cid * _NR + base, _RPT)])


def _make_sc_agg(with_deg):
    mesh = plsc.VectorSubcoreMesh(core_axis_name="c", subcore_axis_name="s")
    outs = [jax.ShapeDtypeStruct((_NC, _NR, _D), jnp.float32)]
    scratch = [
        pltpu.VMEM((2, _IB, _CHUNK), jnp.int32),
        pltpu.VMEM((_IB, _CHUNK), jnp.int32),
        [pltpu.VMEM((_CHUNK, _D), jnp.float32) for _ in range(_NB)],
        [pltpu.SemaphoreType.DMA for _ in range(_NB)],
        pltpu.VMEM_SHARED((_NR, _D), jnp.float32),
    ]
    if with_deg:
        outs.append(jax.ShapeDtypeStruct((_NC * _NR,), jnp.float32))
        scratch += [
            pltpu.VMEM((_CHUNK,), jnp.float32),
            pltpu.VMEM_SHARED((_NR,), jnp.float32),
            pltpu.VMEM((_RPT,), jnp.float32),
        ]

        def body(src_hbm, dst_hbm, p_hbm, out_hbm, deg_hbm,
                 sblk, dblk, bufs, sems, acc, ones, dacc, dbuf):
            _sc_agg_body(True, src_hbm, dst_hbm, p_hbm, out_hbm, deg_hbm,
                         sblk, dblk, bufs, sems, acc, ones, dacc, dbuf)
    else:
        def body(src_hbm, dst_hbm, p_hbm, out_hbm,
                 sblk, dblk, bufs, sems, acc):
            _sc_agg_body(False, src_hbm, dst_hbm, p_hbm, out_hbm, None,
                         sblk, dblk, bufs, sems, acc, None, None, None)

    return pl.kernel(body, out_type=tuple(outs), mesh=mesh,
                     scratch_types=scratch,
                     name="sc_agg_deg" if with_deg else "sc_agg")


_sc_agg_deg = _make_sc_agg(True)
_sc_agg = _make_sc_agg(False)


def _dense_in_body(x_ref, we, be, wl, bl, wr, p_ref, r_ref):
    h = jnp.dot(x_ref[...], we[...], preferred_element_type=jnp.float32)
    h = h + be[...]
    p_ref[...] = jnp.dot(h, wl[...], preferred_element_type=jnp.float32)
    r_ref[...] = jnp.dot(h, wr[...], preferred_element_type=jnp.float32) + bl[...]


def _bn_relu(a0, a1, d0, d1, r, g, b):
    deg = jnp.maximum(d0[...] + d1[...], 1.0)
    u = (a0[...] + a1[...]) / deg + r[...]
    mean = jnp.mean(u, axis=0, keepdims=True)
    var = jnp.mean((u - mean) ** 2, axis=0, keepdims=True)
    hn = (u - mean) * lax.rsqrt(var + 1e-5) * g[...] + b[...]
    return jnp.maximum(hn, 0.0)


def _mid_body(a0, a1, d0, d1, r, g, b, wl, bl, wr, p_ref, r_ref):
    h = _bn_relu(a0, a1, d0, d1, r, g, b)
    p_ref[...] = jnp.dot(h, wl[...], preferred_element_type=jnp.float32)
    r_ref[...] = jnp.dot(h, wr[...], preferred_element_type=jnp.float32) + bl[...]


def _out_body(a0, a1, d0, d1, r, g, b, wc, bc, o_ref):
    h = _bn_relu(a0, a1, d0, d1, r, g, b)
    o_ref[...] = jnp.dot(h, wc[...], preferred_element_type=jnp.float32) + bc[...]


_f = jnp.float32
_dense_in = pl.pallas_call(
    _dense_in_body,
    out_shape=(jax.ShapeDtypeStruct((_N, _D), _f),
               jax.ShapeDtypeStruct((_N, _D), _f)))
_mid = pl.pallas_call(
    _mid_body,
    out_shape=(jax.ShapeDtypeStruct((_N, _D), _f),
               jax.ShapeDtypeStruct((_N, _D), _f)))
_outk = pl.pallas_call(
    _out_body,
    out_shape=jax.ShapeDtypeStruct((_N, 40), _f))


def kernel(x, edge_index, W_emb, b_emb, Wl1, bl1, Wr1, gamma1, beta1,
           Wl2, bl2, Wr2, gamma2, beta2, Wl3, bl3, Wr3, gamma3, beta3,
           W_cls, b_cls):
    src = edge_index[0].astype(jnp.int32)
    dst = edge_index[1].astype(jnp.int32)
    pad = _EP - _E
    src_p = jnp.concatenate([src, jnp.zeros((pad,), jnp.int32)])
    dst_p = jnp.concatenate(
        [dst, _N + (jnp.arange(pad, dtype=jnp.int32) % (_NR - _N))])
    src_p = src_p.reshape(_ROWS, _CHUNK)
    dst_p = dst_p.reshape(_ROWS, _CHUNK)

    p1, r1 = _dense_in(x, W_emb, b_emb, Wl1, bl1, Wr1)
    parts1, degp = _sc_agg_deg(src_p, dst_p, p1)
    d0 = degp[:_N].reshape(_N, 1)
    d1 = degp[_NR:_NR + _N].reshape(_N, 1)

    p2, r2 = _mid(parts1[0, :_N], parts1[1, :_N], d0, d1, r1,
                  gamma1, beta1, Wl2, bl2, Wr2)
    parts2, = _sc_agg(src_p, dst_p, p2)
    p3, r3 = _mid(parts2[0, :_N], parts2[1, :_N], d0, d1, r2,
                  gamma2, beta2, Wl3, bl3, Wr3)
    parts3, = _sc_agg(src_p, dst_p, p3)
    logits = _outk(parts3[0, :_N], parts3[1, :_N], d0, d1, r3,
                   gamma3, beta3, W_cls, b_cls)
    return logits
